# Initial kernel scaffold; baseline (speedup 1.0000x reference)
import jax
import jax.numpy as jnp
from jax.experimental import pallas as pl


def kernel(x, edge_index, edge_attr, u, batch, W1, b1, W2, b2, W3, b3):
    raise NotImplementedError("write your pallas kernel here")


# speedup vs baseline: 2.6022x; 2.6022x over previous
import functools

import jax
import jax.numpy as jnp
from jax import lax
from jax.experimental import pallas as pl
from jax.experimental.pallas import tpu as pltpu
from jax.experimental.pallas import tpu_sc as plsc

N = 10000
D = 256
E = 160000

NC = 2
NS = 16
L = 16
NT = NC * NS

DH = D // NC
CHUNK = 64
NCHT = E // CHUNK
WCH = 40
NWT = N // WCH
WPT = -(-NWT // NS)


def _mesh():
    return plsc.VectorSubcoreMesh(core_axis_name="c", subcore_axis_name="s",
                                  num_cores=NC, num_subcores=NS)


@functools.lru_cache(maxsize=1)
def _make_sc_scatter():
    return pl.kernel(
        _sc_scatter_body,
        out_type=jax.ShapeDtypeStruct((N, D), jnp.float32),
        mesh=_mesh(),
        scratch_types=(
            pltpu.VMEM((CHUNK,), jnp.int32),
            pltpu.VMEM((CHUNK, DH), jnp.float32),
            pltpu.VMEM_SHARED((N, DH), jnp.float32),
        ),
    )


def _sc_scatter_body(edge_hbm, col_hbm, agg_out, idx_v, data_v, acc_sh):
    c = lax.axis_index("c")
    s = lax.axis_index("s")

    z16 = jnp.zeros((L,), jnp.float32)

    def _zero_data(i, _):
        for j in range(DH // L):
            data_v[i, pl.ds(j * L, L)] = z16
        return 0
    lax.fori_loop(0, CHUNK, _zero_data, 0)

    for k in range(WPT):
        w = k * NS + s

        @pl.when(w < NWT)
        def _():
            pltpu.sync_copy(data_v.at[pl.ds(0, WCH)],
                            acc_sh.at[pl.ds(w * WCH, WCH)])
    plsc.subcore_barrier()

    def _chunk(j, _):
        base = (j * NS + s) * CHUNK
        pltpu.sync_copy(col_hbm.at[pl.ds(base, CHUNK)], idx_v)
        pltpu.sync_copy(edge_hbm.at[pl.ds(base, CHUNK), pl.ds(c * DH, DH)],
                        data_v)
        pltpu.sync_copy(data_v, acc_sh.at[idx_v], add=True)
        return 0
    nch = (NCHT - s + NS - 1) // NS
    lax.fori_loop(0, nch, _chunk, 0)
    plsc.subcore_barrier()

    for k in range(WPT):
        w = k * NS + s

        @pl.when(w < NWT)
        def _():
            r0 = w * WCH
            pltpu.sync_copy(acc_sh.at[pl.ds(r0, WCH)], data_v.at[pl.ds(0, WCH)])
            pltpu.sync_copy(data_v.at[pl.ds(0, WCH)],
                            agg_out.at[pl.ds(r0, WCH), pl.ds(c * DH, DH)])


@functools.lru_cache(maxsize=1)
def _make_sc_count():
    return pl.kernel(
        _sc_count_body,
        out_type=jax.ShapeDtypeStruct((NC * N, DH), jnp.float32),
        mesh=_mesh(),
        scratch_types=(
            pltpu.VMEM((CHUNK,), jnp.int32),
            pltpu.VMEM((WCH, DH), jnp.float32),
            pltpu.VMEM((CHUNK, DH), jnp.float32),
            pltpu.VMEM_SHARED((N, DH), jnp.float32),
        ),
    )


def _sc_count_body(col_hbm, cntp_out, idx_v, stage_v, ones_v, cnt_sh):
    c = lax.axis_index("c")
    s = lax.axis_index("s")
    wid = s * NC + c

    z16 = jnp.zeros((L,), jnp.float32)
    ones16 = jnp.ones((L,), jnp.float32)

    def _zero_stage(i, _):
        for j in range(DH // L):
            stage_v[i, pl.ds(j * L, L)] = z16
        return 0
    lax.fori_loop(0, WCH, _zero_stage, 0)

    def _fill_ones(i, _):
        for j in range(DH // L):
            ones_v[i, pl.ds(j * L, L)] = ones16
        return 0
    lax.fori_loop(0, CHUNK, _fill_ones, 0)

    for k in range(WPT):
        w = k * NS + s

        @pl.when(w < NWT)
        def _():
            pltpu.sync_copy(stage_v, cnt_sh.at[pl.ds(w * WCH, WCH)])
    plsc.subcore_barrier()

    def _chunk(j, _):
        base = (j * NT + wid) * CHUNK
        pltpu.sync_copy(col_hbm.at[pl.ds(base, CHUNK)], idx_v)
        pltpu.sync_copy(ones_v, cnt_sh.at[idx_v], add=True)
        return 0
    nch = (NCHT - wid + NT - 1) // NT
    lax.fori_loop(0, nch, _chunk, 0)
    plsc.subcore_barrier()

    for k in range(WPT):
        w = k * NS + s

        @pl.when(w < NWT)
        def _():
            r0 = w * WCH
            pltpu.sync_copy(cnt_sh.at[pl.ds(r0, WCH)], stage_v)
            pltpu.sync_copy(stage_v, cntp_out.at[pl.ds(c * N + r0, WCH)])


RB = 1000


def _mlp_body(x_ref, agg_ref, cntp_ref, w1a_ref, w1b_ref, b1_ref,
              w2_ref, b2_ref, w3_ref, b3_ref, o_ref):
    cnt = cntp_ref[0, :, 0:1] + cntp_ref[1, :, 0:1]
    agg = agg_ref[...] * (1.0 / jnp.maximum(cnt, 1.0))
    h = jnp.dot(x_ref[...], w1a_ref[...], preferred_element_type=jnp.float32)
    h = h + jnp.dot(agg, w1b_ref[...], preferred_element_type=jnp.float32)
    h = jnp.maximum(h + b1_ref[...], 0.0)
    h = jnp.dot(h, w2_ref[...], preferred_element_type=jnp.float32)
    h = jnp.maximum(h + b2_ref[...], 0.0)
    h = jnp.dot(h, w3_ref[...], preferred_element_type=jnp.float32)
    o_ref[...] = h + b3_ref[...]


_tc_mlp = pl.pallas_call(
    _mlp_body,
    grid=(N // RB,),
    in_specs=[
        pl.BlockSpec((RB, D), lambda i: (i, 0)),
        pl.BlockSpec((RB, D), lambda i: (i, 0)),
        pl.BlockSpec((NC, RB, DH), lambda i: (0, i, 0)),
        pl.BlockSpec((D, D), lambda i: (0, 0)),
        pl.BlockSpec((D, D), lambda i: (0, 0)),
        pl.BlockSpec((1, D), lambda i: (0, 0)),
        pl.BlockSpec((D, D), lambda i: (0, 0)),
        pl.BlockSpec((1, D), lambda i: (0, 0)),
        pl.BlockSpec((D, D), lambda i: (0, 0)),
        pl.BlockSpec((1, D), lambda i: (0, 0)),
    ],
    out_specs=pl.BlockSpec((RB, D), lambda i: (i, 0)),
    out_shape=jax.ShapeDtypeStruct((N, D), jnp.float32),
)


@jax.jit
def kernel(x, edge_index, edge_attr, u, batch, W1, b1, W2, b2, W3, b3):
    col = edge_index[1].astype(jnp.int32)
    agg_sum = _make_sc_scatter()(edge_attr, col)
    cntp = _make_sc_count()(col)
    return _tc_mlp(x, agg_sum, cntp.reshape(NC, N, DH), W1[:D], W1[D:],
                   b1.reshape(1, D), W2, b2.reshape(1, D), W3,
                   b3.reshape(1, D))


# speedup vs baseline: 4.0561x; 1.5587x over previous
import functools

import jax
import jax.numpy as jnp
from jax import lax
from jax.experimental import pallas as pl
from jax.experimental.pallas import tpu as pltpu
from jax.experimental.pallas import tpu_sc as plsc

N = 10000
D = 256
E = 160000

NC = 2
NS = 16
L = 16
NT = NC * NS

DH = D // NC
CHUNK = 64
NCHT = E // CHUNK
WCH = 40
NWT = N // WCH
WPT = -(-NWT // NS)


def _mesh():
    return plsc.VectorSubcoreMesh(core_axis_name="c", subcore_axis_name="s",
                                  num_cores=NC, num_subcores=NS)


GMAX = (-(-NCHT // NS) + 1) // 2


@functools.lru_cache(maxsize=1)
def _make_sc_scatter():
    return pl.kernel(
        _sc_scatter_body,
        out_type=jax.ShapeDtypeStruct((N, D), jnp.float32),
        mesh=_mesh(),
        scratch_types=(
            pltpu.VMEM((2, CHUNK), jnp.int32),
            pltpu.VMEM((2, CHUNK, DH), jnp.float32),
            pltpu.VMEM_SHARED((N, DH), jnp.float32),
            pltpu.SemaphoreType.DMA,
            pltpu.SemaphoreType.DMA,
        ),
    )


def _sc_scatter_body(edge_hbm, col_hbm, agg_out, idx_v, data_v, acc_sh,
                     sem0, sem1):
    c = lax.axis_index("c")
    s = lax.axis_index("s")
    sems = (sem0, sem1)

    z16 = jnp.zeros((L,), jnp.float32)

    def _zero_data(i, _):
        for j in range(DH // L):
            data_v[0, i, pl.ds(j * L, L)] = z16
        return 0
    lax.fori_loop(0, CHUNK, _zero_data, 0)

    for k in range(WPT):
        w = k * NS + s

        @pl.when(w < NWT)
        def _():
            pltpu.sync_copy(data_v.at[0, pl.ds(0, WCH)],
                            acc_sh.at[pl.ds(w * WCH, WCH)])
    plsc.subcore_barrier()

    nch = (NCHT - s + NS - 1) // NS

    def _issue_loads(o, b):
        base = (o * NS + s) * CHUNK
        pltpu.async_copy(col_hbm.at[pl.ds(base, CHUNK)], idx_v.at[b], sems[b])
        pltpu.async_copy(edge_hbm.at[pl.ds(base, CHUNK), pl.ds(c * DH, DH)],
                         data_v.at[b], sems[b])

    def _wait_loads(b):
        pltpu.make_async_copy(col_hbm.at[pl.ds(0, CHUNK)], idx_v.at[b],
                              sems[b]).wait()
        pltpu.make_async_copy(edge_hbm.at[pl.ds(0, CHUNK), pl.ds(0, DH)],
                              data_v.at[b], sems[b]).wait()

    @pl.when(0 < nch)
    def _():
        _issue_loads(0, 0)

    def _body(g, _):
        for b in range(2):
            o = g * 2 + b

            @pl.when(o < nch)
            def _():
                _wait_loads(b)

                @pl.when(o + 1 < nch)
                def _():
                    _issue_loads(o + 1, b ^ 1)
                pltpu.sync_copy(data_v.at[b], acc_sh.at[idx_v.at[b]],
                                add=True)
        return 0
    lax.fori_loop(0, GMAX, _body, 0)
    plsc.subcore_barrier()

    for k in range(WPT):
        w = k * NS + s

        @pl.when(w < NWT)
        def _():
            r0 = w * WCH
            pltpu.sync_copy(acc_sh.at[pl.ds(r0, WCH)],
                            data_v.at[0, pl.ds(0, WCH)])
            pltpu.sync_copy(data_v.at[0, pl.ds(0, WCH)],
                            agg_out.at[pl.ds(r0, WCH), pl.ds(c * DH, DH)])


GMAXC = (-(-NCHT // NT) + 1) // 2


@functools.lru_cache(maxsize=1)
def _make_sc_count():
    return pl.kernel(
        _sc_count_body,
        out_type=jax.ShapeDtypeStruct((NC * N, DH), jnp.float32),
        mesh=_mesh(),
        scratch_types=(
            pltpu.VMEM((2, CHUNK), jnp.int32),
            pltpu.VMEM((WCH, DH), jnp.float32),
            pltpu.VMEM((CHUNK, DH), jnp.float32),
            pltpu.VMEM_SHARED((N, DH), jnp.float32),
            pltpu.SemaphoreType.DMA,
            pltpu.SemaphoreType.DMA,
        ),
    )


def _sc_count_body(col_hbm, cntp_out, idx_v, stage_v, ones_v, cnt_sh,
                   sem0, sem1):
    c = lax.axis_index("c")
    s = lax.axis_index("s")
    wid = s * NC + c
    sems = (sem0, sem1)

    z16 = jnp.zeros((L,), jnp.float32)
    ones16 = jnp.ones((L,), jnp.float32)

    def _zero_stage(i, _):
        for j in range(DH // L):
            stage_v[i, pl.ds(j * L, L)] = z16
        return 0
    lax.fori_loop(0, WCH, _zero_stage, 0)

    def _fill_ones(i, _):
        for j in range(DH // L):
            ones_v[i, pl.ds(j * L, L)] = ones16
        return 0
    lax.fori_loop(0, CHUNK, _fill_ones, 0)

    for k in range(WPT):
        w = k * NS + s

        @pl.when(w < NWT)
        def _():
            pltpu.sync_copy(stage_v, cnt_sh.at[pl.ds(w * WCH, WCH)])
    plsc.subcore_barrier()

    nch = (NCHT - wid + NT - 1) // NT

    def _issue_idx(o, b):
        base = (o * NT + wid) * CHUNK
        pltpu.async_copy(col_hbm.at[pl.ds(base, CHUNK)], idx_v.at[b], sems[b])

    def _wait_idx(b):
        pltpu.make_async_copy(col_hbm.at[pl.ds(0, CHUNK)], idx_v.at[b],
                              sems[b]).wait()

    @pl.when(0 < nch)
    def _():
        _issue_idx(0, 0)

    def _body(g, _):
        for b in range(2):
            o = g * 2 + b

            @pl.when(o < nch)
            def _():
                _wait_idx(b)

                @pl.when(o + 1 < nch)
                def _():
                    _issue_idx(o + 1, b ^ 1)
                pltpu.sync_copy(ones_v, cnt_sh.at[idx_v.at[b]], add=True)
        return 0
    lax.fori_loop(0, GMAXC, _body, 0)
    plsc.subcore_barrier()

    for k in range(WPT):
        w = k * NS + s

        @pl.when(w < NWT)
        def _():
            r0 = w * WCH
            pltpu.sync_copy(cnt_sh.at[pl.ds(r0, WCH)], stage_v)
            pltpu.sync_copy(stage_v, cntp_out.at[pl.ds(c * N + r0, WCH)])


RB = 1000


def _mlp_body(x_ref, agg_ref, cntp_ref, w1a_ref, w1b_ref, b1_ref,
              w2_ref, b2_ref, w3_ref, b3_ref, o_ref):
    cnt = cntp_ref[0, :, 0:1] + cntp_ref[1, :, 0:1]
    agg = agg_ref[...] * (1.0 / jnp.maximum(cnt, 1.0))
    h = jnp.dot(x_ref[...], w1a_ref[...], preferred_element_type=jnp.float32)
    h = h + jnp.dot(agg, w1b_ref[...], preferred_element_type=jnp.float32)
    h = jnp.maximum(h + b1_ref[...], 0.0)
    h = jnp.dot(h, w2_ref[...], preferred_element_type=jnp.float32)
    h = jnp.maximum(h + b2_ref[...], 0.0)
    h = jnp.dot(h, w3_ref[...], preferred_element_type=jnp.float32)
    o_ref[...] = h + b3_ref[...]


_tc_mlp = pl.pallas_call(
    _mlp_body,
    grid=(N // RB,),
    in_specs=[
        pl.BlockSpec((RB, D), lambda i: (i, 0)),
        pl.BlockSpec((RB, D), lambda i: (i, 0)),
        pl.BlockSpec((NC, RB, DH), lambda i: (0, i, 0)),
        pl.BlockSpec((D, D), lambda i: (0, 0)),
        pl.BlockSpec((D, D), lambda i: (0, 0)),
        pl.BlockSpec((1, D), lambda i: (0, 0)),
        pl.BlockSpec((D, D), lambda i: (0, 0)),
        pl.BlockSpec((1, D), lambda i: (0, 0)),
        pl.BlockSpec((D, D), lambda i: (0, 0)),
        pl.BlockSpec((1, D), lambda i: (0, 0)),
    ],
    out_specs=pl.BlockSpec((RB, D), lambda i: (i, 0)),
    out_shape=jax.ShapeDtypeStruct((N, D), jnp.float32),
)


@jax.jit
def kernel(x, edge_index, edge_attr, u, batch, W1, b1, W2, b2, W3, b3):
    col = edge_index[1].astype(jnp.int32)
    agg_sum = _make_sc_scatter()(edge_attr, col)
    cntp = _make_sc_count()(col)
    return _tc_mlp(x, agg_sum, cntp.reshape(NC, N, DH), W1[:D], W1[D:],
                   b1.reshape(1, D), W2, b2.reshape(1, D), W3,
                   b3.reshape(1, D))
